# baseline (speedup 1.0000x reference)
import functools

import jax
import jax.numpy as jnp
import jax.scipy.linalg as jsl
from jax import lax
from jax.experimental import pallas as pl
from jax.experimental.pallas import tpu as pltpu
from jax.experimental.pallas import tpu_sc as plsc

N = 10000
E = 320000
D = 128
R = 8
NR = 64
NRANGES = 160
NPAD = NRANGES * NR
NTILES = 32
RPT = NRANGES // NTILES
EPT = E // NTILES
CAP = 10240
CE = 2000
CHUNK = 128
M = R * NR
DUMMY = M
MA = M + 16
NRP = 176
STG = EPT + NRANGES * (CHUNK - 1) + 32
NEG = float("-inf")


def _wid():
    return lax.axis_index("s") * 2 + lax.axis_index("c")


def _iota16():
    return lax.iota(jnp.int32, 16)


def _sstore(ref, i, val):
    b = pl.multiple_of(jnp.left_shift(jnp.right_shift(i, 3), 3), 8)
    w = ref[pl.ds(b, 16)]
    ref[pl.ds(b, 16)] = jnp.where(_iota16() == (i - b), val, w)


def _bin_kernel_def():
  return functools.partial(
    pl.kernel,
    mesh=plsc.VectorSubcoreMesh(core_axis_name="c", subcore_axis_name="s"),
    out_type=[
        jax.ShapeDtypeStruct((NRANGES, NTILES, CAP), jnp.int32),
        jax.ShapeDtypeStruct((NRANGES, NTILES, 16), jnp.int32),
    ],
    scratch_types=[
        pltpu.VMEM((CE,), jnp.int32),
        pltpu.VMEM((CE,), jnp.int32),
        pltpu.VMEM((CE,), jnp.int32),
        pltpu.VMEM((EPT,), jnp.int32),
        pltpu.VMEM((EPT,), jnp.int32),
        pltpu.SMEM((NRANGES,), jnp.int32),
        pltpu.SMEM((NRANGES,), jnp.int32),
        pltpu.SMEM((NRANGES,), jnp.int32),
        pltpu.VMEM((16,), jnp.int32),
        pltpu.VMEM((STG,), jnp.int32),
    ],
  )


def _bin_edges_body(src_h, dst_h, et_h, lists_h, hdr_h,
                    sbuf, dbuf, tbuf, ridb, pkb, cntv, offv, curv, hb, stg):
    t = _wid()
    base_e = pl.multiple_of(t * EPT, 8)

    def z_body(r, _):
        cntv[r] = 0
        return 0

    lax.fori_loop(0, NRANGES, z_body, 0)

    dum = jnp.full((16,), DUMMY, jnp.int32)

    def stg_body(v, _):
        stg[pl.ds(pl.multiple_of(v * 16, 16), 16)] = dum
        return 0

    lax.fori_loop(0, STG // 16, stg_body, 0)

    for c in range(EPT // CE):
        pltpu.sync_copy(src_h.at[pl.ds(base_e + c * CE, CE)], sbuf)
        pltpu.sync_copy(dst_h.at[pl.ds(base_e + c * CE, CE)], dbuf)
        pltpu.sync_copy(et_h.at[pl.ds(base_e + c * CE, CE)], tbuf)

        def v_body(v, _):
            vb = pl.multiple_of(v * 16, 16)
            d = dbuf[pl.ds(vb, 16)]
            s = sbuf[pl.ds(vb, 16)]
            ty = tbuf[pl.ds(vb, 16)]
            rid = jnp.right_shift(d, 6)
            pk = jnp.left_shift(s, 10) | jnp.left_shift(ty, 6) | (d & 63)
            g = pl.multiple_of(c * CE + vb, 16)
            ridb[pl.ds(g, 16)] = rid
            pkb[pl.ds(g, 16)] = pk
            return 0

        lax.fori_loop(0, CE // 16, v_body, 0)

    def cnt_body(v, _):
        vb = pl.multiple_of(v * 16, 16)
        rid = ridb[pl.ds(vb, 16)]
        for l in range(16):
            r = rid[l]
            cntv[r] = cntv[r] + 1
        return 0

    lax.fori_loop(0, EPT // 16, cnt_body, 0)

    def pfx_body(r, cum):
        offv[r] = cum
        curv[r] = cum
        return cum + ((cntv[r] + CHUNK - 1) & ~(CHUNK - 1))

    lax.fori_loop(0, NRANGES, pfx_body, jnp.int32(0))

    def app_body(v, _):
        vb = pl.multiple_of(v * 16, 16)
        rid = ridb[pl.ds(vb, 16)]
        pk = pkb[pl.ds(vb, 16)]
        for l in range(16):
            r = rid[l]
            o = curv[r]
            _sstore(stg, o, pk[l])
            curv[r] = o + 1
        return 0

    lax.fori_loop(0, EPT // 16, app_body, 0)

    def wr_body(r, _):
        cnt = cntv[r]
        o = offv[r]
        nch = (cnt + CHUNK - 1) // CHUNK

        def ch_body(c, _2):
            so = pl.multiple_of(o + c * CHUNK, CHUNK)
            do = pl.multiple_of(c * CHUNK, CHUNK)
            pltpu.sync_copy(stg.at[pl.ds(so, CHUNK)],
                            lists_h.at[r, t, pl.ds(do, CHUNK)])
            return 0

        lax.fori_loop(0, nch, ch_body, 0)
        hb[pl.ds(0, 16)] = jnp.where(_iota16() == 0, cnt, 0)
        pltpu.sync_copy(hb, hdr_h.at[r, t])
        return 0

    lax.fori_loop(0, NRANGES, wr_body, 0)


def _segmax_kernel_def():
  return functools.partial(
    pl.kernel,
    mesh=plsc.VectorSubcoreMesh(core_axis_name="c", subcore_axis_name="s"),
    out_type=jax.ShapeDtypeStruct((NRANGES, M, D), jnp.float32),
    scratch_types=[
        pltpu.VMEM((MA, D), jnp.float32),
        pltpu.VMEM((CHUNK, D), jnp.float32),
        pltpu.VMEM((NTILES, 16), jnp.int32),
        pltpu.VMEM((CHUNK,), jnp.int32),
        pltpu.VMEM((CHUNK,), jnp.int32),
        pltpu.SemaphoreType.DMA,
    ],
  )


def _segmax_body(table_h, lists_h, hdr_h, hall_h,
                 acc, msg, hdrv, pkb, idxb, sem):
    t = _wid()
    neg = jnp.full((16,), NEG, jnp.float32)

    for k in range(RPT):
        j = t + NTILES * k

        def init_body(i, _):
            for f in range(D // 16):
                acc[i, pl.ds(f * 16, 16)] = neg
            return 0

        lax.fori_loop(0, MA, init_body, 0)
        pltpu.sync_copy(hdr_h.at[j], hdrv)

        def tile_body(tp, _):
            cnt = hdrv[tp, pl.ds(0, 16)][0]
            nch = (cnt + CHUNK - 1) // CHUNK

            def ch_body(c, _2):
                cb = pl.multiple_of(c * CHUNK, CHUNK)
                pltpu.sync_copy(lists_h.at[j, tp, pl.ds(cb, CHUNK)], pkb)
                for v in range(CHUNK // 16):
                    idxb[pl.ds(v * 16, 16)] = jnp.right_shift(
                        pkb[pl.ds(v * 16, 16)], 10)
                pltpu.async_copy(table_h.at[idxb], msg, sem).wait()
                rem = jnp.minimum(cnt - cb, CHUNK)
                nv = jnp.right_shift(rem + 15, 4)

                def v_body(v, _3):
                    vb = pl.multiple_of(v * 16, 16)
                    pkv = pkb[pl.ds(vb, 16)]
                    for l in range(16):
                        m = pkv[l] & 1023
                        for f in range(D // 16):
                            sl = pl.ds(f * 16, 16)
                            acc[m, sl] = jnp.maximum(acc[m, sl],
                                                     msg[vb + l, sl])
                    return 0

                lax.fori_loop(0, nv, v_body, 0)
                return 0

            lax.fori_loop(0, nch, ch_body, 0)
            return 0

        lax.fori_loop(0, NTILES, tile_body, 0)
        pltpu.sync_copy(acc.at[pl.ds(0, M)], hall_h.at[j])


def _tc1_body(x_ref, hall_ref, root_ref, bias_ref, w_ref, out_ref):
    acc = jnp.dot(x_ref[...], root_ref[...],
                  preferred_element_type=jnp.float32) + bias_ref[...]
    hb = hall_ref[0]
    for r in range(R):
        h = hb[r * NR:(r + 1) * NR, :]
        h = jnp.where(h == NEG, 0.0, h)
        acc = acc + jnp.dot(h, w_ref[r], preferred_element_type=jnp.float32)
    out_ref[...] = jnp.maximum(acc, 0.0)


def _tc_layer1(xp, hall, root1, bias1, w1bd):
    return pl.pallas_call(
        _tc1_body,
        grid=(NRANGES,),
        in_specs=[
            pl.BlockSpec((NR, D), lambda j: (j, 0)),
            pl.BlockSpec((1, M, D), lambda j: (j, 0, 0)),
            pl.BlockSpec((D, D), lambda j: (0, 0)),
            pl.BlockSpec((1, D), lambda j: (0, 0)),
            pl.BlockSpec((R, D, D), lambda j: (0, 0, 0)),
        ],
        out_specs=pl.BlockSpec((NR, D), lambda j: (j, 0)),
        out_shape=jax.ShapeDtypeStruct((NPAD, D), jnp.float32),
    )(xp, hall, root1, bias1, w1bd)


def _tc2_body(h1_ref, hall_ref, root_ref, bias_ref, comp_ref, basis_ref,
              out_ref):
    acc = jnp.dot(h1_ref[...], root_ref[...],
                  preferred_element_type=jnp.float32) + bias_ref[...]
    hb = hall_ref[0]
    hfix = [None] * R
    for r in range(R):
        h = hb[r * NR:(r + 1) * NR, :]
        hfix[r] = jnp.where(h == NEG, 0.0, h)
    for b in range(4):
        g = hfix[0] * comp_ref[0, b]
        for r in range(1, R):
            g = g + hfix[r] * comp_ref[r, b]
        acc = acc + jnp.dot(g, basis_ref[b],
                            preferred_element_type=jnp.float32)
    out_ref[...] = acc


def _tc_layer2(h1, hall, root2, bias2, comp2, basis2):
    return pl.pallas_call(
        _tc2_body,
        grid=(NRANGES,),
        in_specs=[
            pl.BlockSpec((NR, D), lambda j: (j, 0)),
            pl.BlockSpec((1, M, D), lambda j: (j, 0, 0)),
            pl.BlockSpec((D, 2), lambda j: (0, 0)),
            pl.BlockSpec((1, 2), lambda j: (0, 0)),
            pl.BlockSpec((R, 4), lambda j: (0, 0)),
            pl.BlockSpec((4, D, 2), lambda j: (0, 0, 0)),
        ],
        out_specs=pl.BlockSpec((NR, 2), lambda j: (j, 0)),
        out_shape=jax.ShapeDtypeStruct((NPAD, 2), jnp.float32),
    )(h1, hall, root2, bias2, comp2, basis2)


@functools.cache
def _sc_kernels():
    bin_edges = _bin_kernel_def()(_bin_edges_body)
    segmax = _segmax_kernel_def()(_segmax_body)
    return bin_edges, segmax


def kernel(x, edge_index, edge_type, weight1, root1, bias1, comp2, basis2,
           root2, bias2):
    _bin_edges, _segmax = _sc_kernels()
    src = edge_index[0]
    dst = edge_index[1]
    et = edge_type.astype(jnp.int32)
    xp = jnp.pad(x, ((0, NPAD - N), (0, 0)))
    w1bd = jax.vmap(
        lambda w: jsl.block_diag(w[0], w[1], w[2], w[3]))(weight1)

    lists, hdr = _bin_edges(src, dst, et)
    hall1 = _segmax(xp, lists, hdr)
    h1 = _tc_layer1(xp, hall1, root1, bias1.reshape(1, D), w1bd)
    hall2 = _segmax(h1, lists, hdr)
    out = _tc_layer2(h1, hall2, root2, bias2.reshape(1, 2), comp2, basis2)
    return out[:N]


# speedup vs baseline: 2.0847x; 2.0847x over previous
import functools

import jax
import jax.numpy as jnp
import jax.scipy.linalg as jsl
from jax import lax
from jax.experimental import pallas as pl
from jax.experimental.pallas import tpu as pltpu
from jax.experimental.pallas import tpu_sc as plsc

N = 10000
E = 320000
D = 128
R = 8
NR = 64
NRANGES = 160
NPAD = NRANGES * NR
NTILES = 32
RPT = NRANGES // NTILES
EPT = E // NTILES
CAP = 10240
CE = 2000
CHUNK = 64
M = R * NR
DUMMY = M
MA = M + 16
NRP = 176
STG = EPT + NRANGES * (CHUNK - 1) + 32
NEG = float("-inf")


def _wid():
    return lax.axis_index("s") * 2 + lax.axis_index("c")


def _iota16():
    return lax.iota(jnp.int32, 16)


def _sstore(ref, i, val):
    b = pl.multiple_of(jnp.left_shift(jnp.right_shift(i, 3), 3), 8)
    w = ref[pl.ds(b, 16)]
    ref[pl.ds(b, 16)] = jnp.where(_iota16() == (i - b), val, w)


def _bin_kernel_def():
  return functools.partial(
    pl.kernel,
    mesh=plsc.VectorSubcoreMesh(core_axis_name="c", subcore_axis_name="s"),
    out_type=[
        jax.ShapeDtypeStruct((NRANGES, NTILES, CAP), jnp.int32),
        jax.ShapeDtypeStruct((NRANGES, NTILES, 16), jnp.int32),
    ],
    scratch_types=[
        pltpu.VMEM((CE,), jnp.int32),
        pltpu.VMEM((CE,), jnp.int32),
        pltpu.VMEM((CE,), jnp.int32),
        pltpu.VMEM((EPT,), jnp.int32),
        pltpu.VMEM((EPT,), jnp.int32),
        pltpu.SMEM((NRANGES,), jnp.int32),
        pltpu.SMEM((NRANGES,), jnp.int32),
        pltpu.SMEM((NRANGES,), jnp.int32),
        pltpu.VMEM((16,), jnp.int32),
        pltpu.VMEM((STG,), jnp.int32),
    ],
  )


def _bin_edges_body(src_h, dst_h, et_h, lists_h, hdr_h,
                    sbuf, dbuf, tbuf, ridb, pkb, cntv, offv, curv, hb, stg):
    t = _wid()
    base_e = pl.multiple_of(t * EPT, 8)

    def z_body(r, _):
        cntv[r] = 0
        return 0

    lax.fori_loop(0, NRANGES, z_body, 0)

    dum = jnp.full((16,), DUMMY, jnp.int32)

    def stg_body(v, _):
        stg[pl.ds(pl.multiple_of(v * 16, 16), 16)] = dum
        return 0

    lax.fori_loop(0, STG // 16, stg_body, 0)

    for c in range(EPT // CE):
        pltpu.sync_copy(src_h.at[pl.ds(base_e + c * CE, CE)], sbuf)
        pltpu.sync_copy(dst_h.at[pl.ds(base_e + c * CE, CE)], dbuf)
        pltpu.sync_copy(et_h.at[pl.ds(base_e + c * CE, CE)], tbuf)

        def v_body(v, _):
            vb = pl.multiple_of(v * 16, 16)
            d = dbuf[pl.ds(vb, 16)]
            s = sbuf[pl.ds(vb, 16)]
            ty = tbuf[pl.ds(vb, 16)]
            rid = jnp.right_shift(d, 6)
            pk = jnp.left_shift(s, 10) | jnp.left_shift(ty, 6) | (d & 63)
            g = pl.multiple_of(c * CE + vb, 16)
            ridb[pl.ds(g, 16)] = rid
            pkb[pl.ds(g, 16)] = pk
            return 0

        lax.fori_loop(0, CE // 16, v_body, 0)

    def cnt_body(v, _):
        vb = pl.multiple_of(v * 16, 16)
        rid = ridb[pl.ds(vb, 16)]
        for l in range(16):
            r = rid[l]
            cntv[r] = cntv[r] + 1
        return 0

    lax.fori_loop(0, EPT // 16, cnt_body, 0)

    def pfx_body(r, cum):
        offv[r] = cum
        curv[r] = cum
        return cum + ((cntv[r] + CHUNK - 1) & ~(CHUNK - 1))

    lax.fori_loop(0, NRANGES, pfx_body, jnp.int32(0))

    def app_body(v, _):
        vb = pl.multiple_of(v * 16, 16)
        rid = ridb[pl.ds(vb, 16)]
        pk = pkb[pl.ds(vb, 16)]
        for l in range(16):
            r = rid[l]
            o = curv[r]
            _sstore(stg, o, pk[l])
            curv[r] = o + 1
        return 0

    lax.fori_loop(0, EPT // 16, app_body, 0)

    def wr_body(r, _):
        cnt = cntv[r]
        o = offv[r]
        nch = (cnt + CHUNK - 1) // CHUNK

        def ch_body(c, _2):
            so = pl.multiple_of(o + c * CHUNK, CHUNK)
            do = pl.multiple_of(c * CHUNK, CHUNK)
            pltpu.sync_copy(stg.at[pl.ds(so, CHUNK)],
                            lists_h.at[r, t, pl.ds(do, CHUNK)])
            return 0

        lax.fori_loop(0, nch, ch_body, 0)
        hb[pl.ds(0, 16)] = jnp.where(_iota16() == 0, cnt, 0)
        pltpu.sync_copy(hb, hdr_h.at[r, t])
        return 0

    lax.fori_loop(0, NRANGES, wr_body, 0)


def _segmax_kernel_def():
  return functools.partial(
    pl.kernel,
    mesh=plsc.VectorSubcoreMesh(core_axis_name="c", subcore_axis_name="s"),
    out_type=jax.ShapeDtypeStruct((NRANGES, M, D), jnp.float32),
    scratch_types=[
        pltpu.VMEM((MA, D), jnp.float32),
        pltpu.VMEM((CHUNK, D), jnp.float32),
        pltpu.VMEM((NTILES, 16), jnp.int32),
        pltpu.VMEM((CHUNK,), jnp.int32),
        pltpu.VMEM((CHUNK,), jnp.int32),
        pltpu.SemaphoreType.DMA,
    ],
  )


def _segmax_body(table_h, lists_h, hdr_h, hall_h,
                 acc, msg, hdrv, pkb, idxb, sem):
    t = _wid()
    neg = jnp.full((16,), NEG, jnp.float32)

    for k in range(RPT):
        j = t + NTILES * k

        def init_body(i, _):
            for f in range(D // 16):
                acc[i, pl.ds(f * 16, 16)] = neg
            return 0

        lax.fori_loop(0, MA, init_body, 0)
        pltpu.sync_copy(hdr_h.at[j], hdrv)

        def tile_body(tp, _):
            cnt = hdrv[tp, pl.ds(0, 16)][0]
            nch = (cnt + CHUNK - 1) // CHUNK

            def ch_body(c, _2):
                cb = pl.multiple_of(c * CHUNK, CHUNK)
                pltpu.sync_copy(lists_h.at[j, tp, pl.ds(cb, CHUNK)], pkb)
                for v in range(CHUNK // 16):
                    idxb[pl.ds(v * 16, 16)] = jnp.right_shift(
                        pkb[pl.ds(v * 16, 16)], 10)
                pltpu.async_copy(table_h.at[idxb], msg, sem).wait()
                rem = jnp.minimum(cnt - cb, CHUNK)
                nv = jnp.right_shift(rem + 15, 4)

                def v_body(v, _3):
                    vb = pl.multiple_of(v * 16, 16)
                    pkv = pkb[pl.ds(vb, 16)]
                    for l in range(16):
                        m = pkv[l] & 1023
                        for f in range(D // 16):
                            sl = pl.ds(f * 16, 16)
                            acc[m, sl] = jnp.maximum(acc[m, sl],
                                                     msg[vb + l, sl])
                    return 0

                lax.fori_loop(0, nv, v_body, 0)
                return 0

            lax.fori_loop(0, nch, ch_body, 0)
            return 0

        lax.fori_loop(0, NTILES, tile_body, 0)
        pltpu.sync_copy(acc.at[pl.ds(0, M)], hall_h.at[j])


def _tc1_body(x_ref, hall_ref, root_ref, bias_ref, w_ref, out_ref):
    acc = jnp.dot(x_ref[...], root_ref[...],
                  preferred_element_type=jnp.float32) + bias_ref[...]
    hb = hall_ref[0]
    for r in range(R):
        h = hb[r * NR:(r + 1) * NR, :]
        h = jnp.where(h == NEG, 0.0, h)
        acc = acc + jnp.dot(h, w_ref[r], preferred_element_type=jnp.float32)
    out_ref[...] = jnp.maximum(acc, 0.0)


def _tc_layer1(xp, hall, root1, bias1, w1bd):
    return pl.pallas_call(
        _tc1_body,
        grid=(NRANGES,),
        in_specs=[
            pl.BlockSpec((NR, D), lambda j: (j, 0)),
            pl.BlockSpec((1, M, D), lambda j: (j, 0, 0)),
            pl.BlockSpec((D, D), lambda j: (0, 0)),
            pl.BlockSpec((1, D), lambda j: (0, 0)),
            pl.BlockSpec((R, D, D), lambda j: (0, 0, 0)),
        ],
        out_specs=pl.BlockSpec((NR, D), lambda j: (j, 0)),
        out_shape=jax.ShapeDtypeStruct((NPAD, D), jnp.float32),
    )(xp, hall, root1, bias1, w1bd)


def _tc2_body(h1_ref, hall_ref, root_ref, bias_ref, comp_ref, basis_ref,
              out_ref):
    acc = jnp.dot(h1_ref[...], root_ref[...],
                  preferred_element_type=jnp.float32) + bias_ref[...]
    hb = hall_ref[0]
    hfix = [None] * R
    for r in range(R):
        h = hb[r * NR:(r + 1) * NR, :]
        hfix[r] = jnp.where(h == NEG, 0.0, h)
    for b in range(4):
        g = hfix[0] * comp_ref[0, b]
        for r in range(1, R):
            g = g + hfix[r] * comp_ref[r, b]
        acc = acc + jnp.dot(g, basis_ref[b],
                            preferred_element_type=jnp.float32)
    out_ref[...] = acc


def _tc_layer2(h1, hall, root2, bias2, comp2, basis2):
    return pl.pallas_call(
        _tc2_body,
        grid=(NRANGES,),
        in_specs=[
            pl.BlockSpec((NR, D), lambda j: (j, 0)),
            pl.BlockSpec((1, M, D), lambda j: (j, 0, 0)),
            pl.BlockSpec((D, 2), lambda j: (0, 0)),
            pl.BlockSpec((1, 2), lambda j: (0, 0)),
            pl.BlockSpec((R, 4), lambda j: (0, 0)),
            pl.BlockSpec((4, D, 2), lambda j: (0, 0, 0)),
        ],
        out_specs=pl.BlockSpec((NR, 2), lambda j: (j, 0)),
        out_shape=jax.ShapeDtypeStruct((NPAD, 2), jnp.float32),
    )(h1, hall, root2, bias2, comp2, basis2)


@functools.cache
def _sc_kernels():
    bin_edges = _bin_kernel_def()(_bin_edges_body)
    segmax = _segmax_kernel_def()(_segmax_body)
    return bin_edges, segmax


def kernel(x, edge_index, edge_type, weight1, root1, bias1, comp2, basis2,
           root2, bias2):
    _bin_edges, _segmax = _sc_kernels()
    src = edge_index[0]
    dst = edge_index[1]
    et = edge_type.astype(jnp.int32)
    xp = jnp.pad(x, ((0, NPAD - N), (0, 0)))
    w1bd = jax.vmap(
        lambda w: jsl.block_diag(w[0], w[1], w[2], w[3]))(weight1)

    lists, hdr = _bin_edges(src, dst, et)
    hall1 = _segmax(xp, lists, hdr)
    h1 = _tc_layer1(xp, hall1, root1, bias1.reshape(1, D), w1bd)
    hall2 = _segmax(h1, lists, hdr)
    out = _tc_layer2(h1, hall2, root2, bias2.reshape(1, 2), comp2, basis2)
    return out[:N]


# speedup vs baseline: 2.0851x; 1.0002x over previous
import functools

import jax
import jax.numpy as jnp
import jax.scipy.linalg as jsl
from jax import lax
from jax.experimental import pallas as pl
from jax.experimental.pallas import tpu as pltpu
from jax.experimental.pallas import tpu_sc as plsc

N = 10000
E = 320000
D = 128
R = 8
NR = 64
NRANGES = 160
NPAD = NRANGES * NR
NTILES = 32
RPT = NRANGES // NTILES
EPT = E // NTILES
CAP = 10240
CE = 2000
CHUNK = 64
M = R * NR
DUMMY = M
MA = M + 16
NRP = 176
STG = EPT + NRANGES * (CHUNK - 1) + 32
NEG = float("-inf")


def _wid():
    return lax.axis_index("s") * 2 + lax.axis_index("c")


def _iota16():
    return lax.iota(jnp.int32, 16)


def _sstore(ref, i, val):
    b = pl.multiple_of(jnp.left_shift(jnp.right_shift(i, 3), 3), 8)
    w = ref[pl.ds(b, 16)]
    ref[pl.ds(b, 16)] = jnp.where(_iota16() == (i - b), val, w)


def _bin_kernel_def():
  return functools.partial(
    pl.kernel,
    mesh=plsc.VectorSubcoreMesh(core_axis_name="c", subcore_axis_name="s"),
    out_type=[
        jax.ShapeDtypeStruct((NRANGES, NTILES, CAP), jnp.int32),
        jax.ShapeDtypeStruct((NRANGES, NTILES, 16), jnp.int32),
    ],
    scratch_types=[
        pltpu.VMEM((CE,), jnp.int32),
        pltpu.VMEM((CE,), jnp.int32),
        pltpu.VMEM((CE,), jnp.int32),
        pltpu.VMEM((EPT,), jnp.int32),
        pltpu.VMEM((EPT,), jnp.int32),
        pltpu.SMEM((NRANGES,), jnp.int32),
        pltpu.SMEM((NRANGES,), jnp.int32),
        pltpu.SMEM((NRANGES,), jnp.int32),
        pltpu.VMEM((16,), jnp.int32),
        pltpu.VMEM((STG,), jnp.int32),
    ],
  )


def _bin_edges_body(src_h, dst_h, et_h, lists_h, hdr_h,
                    sbuf, dbuf, tbuf, ridb, pkb, cntv, offv, curv, hb, stg):
    t = _wid()
    base_e = pl.multiple_of(t * EPT, 8)

    def z_body(r, _):
        cntv[r] = 0
        return 0

    lax.fori_loop(0, NRANGES, z_body, 0)

    dum = jnp.full((16,), DUMMY, jnp.int32)

    def stg_body(v, _):
        stg[pl.ds(pl.multiple_of(v * 16, 16), 16)] = dum
        return 0

    lax.fori_loop(0, STG // 16, stg_body, 0)

    for c in range(EPT // CE):
        pltpu.sync_copy(src_h.at[pl.ds(base_e + c * CE, CE)], sbuf)
        pltpu.sync_copy(dst_h.at[pl.ds(base_e + c * CE, CE)], dbuf)
        pltpu.sync_copy(et_h.at[pl.ds(base_e + c * CE, CE)], tbuf)

        def v_body(v, _):
            vb = pl.multiple_of(v * 16, 16)
            d = dbuf[pl.ds(vb, 16)]
            s = sbuf[pl.ds(vb, 16)]
            ty = tbuf[pl.ds(vb, 16)]
            rid = jnp.right_shift(d, 6)
            pk = jnp.left_shift(s, 10) | jnp.left_shift(ty, 6) | (d & 63)
            g = pl.multiple_of(c * CE + vb, 16)
            ridb[pl.ds(g, 16)] = rid
            pkb[pl.ds(g, 16)] = pk
            return 0

        lax.fori_loop(0, CE // 16, v_body, 0)

    def cnt_body(v, _):
        vb = pl.multiple_of(v * 16, 16)
        rid = ridb[pl.ds(vb, 16)]
        for l in range(16):
            r = rid[l]
            cntv[r] = cntv[r] + 1
        return 0

    lax.fori_loop(0, EPT // 16, cnt_body, 0)

    def pfx_body(r, cum):
        offv[r] = cum
        curv[r] = cum
        return cum + ((cntv[r] + CHUNK - 1) & ~(CHUNK - 1))

    lax.fori_loop(0, NRANGES, pfx_body, jnp.int32(0))

    def app_body(v, _):
        vb = pl.multiple_of(v * 16, 16)
        rid = ridb[pl.ds(vb, 16)]
        pk = pkb[pl.ds(vb, 16)]
        for l in range(16):
            r = rid[l]
            o = curv[r]
            _sstore(stg, o, pk[l])
            curv[r] = o + 1
        return 0

    lax.fori_loop(0, EPT // 16, app_body, 0)

    def wr_body(r, _):
        cnt = cntv[r]
        o = offv[r]
        nch = (cnt + CHUNK - 1) // CHUNK

        def ch_body(c, _2):
            so = pl.multiple_of(o + c * CHUNK, CHUNK)
            do = pl.multiple_of(c * CHUNK, CHUNK)
            pltpu.sync_copy(stg.at[pl.ds(so, CHUNK)],
                            lists_h.at[r, t, pl.ds(do, CHUNK)])
            return 0

        lax.fori_loop(0, nch, ch_body, 0)
        hb[pl.ds(0, 16)] = jnp.where(_iota16() == 0, cnt, 0)
        pltpu.sync_copy(hb, hdr_h.at[r, t])
        return 0

    lax.fori_loop(0, NRANGES, wr_body, 0)


def _segmax_kernel_def():
  return functools.partial(
    pl.kernel,
    mesh=plsc.VectorSubcoreMesh(core_axis_name="c", subcore_axis_name="s"),
    out_type=jax.ShapeDtypeStruct((NRANGES, M, D), jnp.float32),
    scratch_types=[
        pltpu.VMEM((MA, D), jnp.float32),
        pltpu.VMEM((CHUNK, D), jnp.float32),
        pltpu.VMEM((CHUNK, D), jnp.float32),
        pltpu.VMEM((NTILES, 16), jnp.int32),
        pltpu.VMEM((CHUNK,), jnp.int32),
        pltpu.VMEM((CHUNK,), jnp.int32),
        pltpu.VMEM((CHUNK,), jnp.int32),
        pltpu.VMEM((CHUNK,), jnp.int32),
        pltpu.SemaphoreType.DMA,
        pltpu.SemaphoreType.DMA,
    ],
  )


def _segmax_body(table_h, lists_h, hdr_h, hall_h,
                 acc, msgA, msgB, hdrv, pkbA, pkbB, idxA, idxB, semA, semB):
    t = _wid()
    neg = jnp.full((16,), NEG, jnp.float32)

    def getcnt(tp):
        return hdrv[tp, pl.ds(0, 16)][0]

    def load_unpack(j, tp, cb, pkb, idxb):
        pltpu.sync_copy(lists_h.at[j, tp, pl.ds(cb, CHUNK)], pkb)
        for v in range(CHUNK // 16):
            idxb[pl.ds(v * 16, 16)] = jnp.right_shift(
                pkb[pl.ds(v * 16, 16)], 10)

    def fire(j, tp, pkb, idxb, msg, sem):
        @pl.when(getcnt(tp) > 0)
        def _():
            load_unpack(j, tp, 0, pkb, idxb)
            pltpu.async_copy(table_h.at[idxb], msg, sem)

    def process(rem, pkb, msg):
        nv = jnp.right_shift(jnp.minimum(rem, CHUNK) + 15, 4)

        def v_body(v, _3):
            vb = pl.multiple_of(v * 16, 16)
            pkv = pkb[pl.ds(vb, 16)]
            for l in range(16):
                m = pkv[l] & 1023
                for f in range(D // 16):
                    sl = pl.ds(f * 16, 16)
                    acc[m, sl] = jnp.maximum(acc[m, sl], msg[vb + l, sl])
            return 0

        lax.fori_loop(0, nv, v_body, 0)

    def waitprocess(j, tp, pkb, idxb, msg, sem):
        cnt = getcnt(tp)

        @pl.when(cnt > 0)
        def _():
            pltpu.make_async_copy(table_h.at[idxb], msg, sem).wait()
            process(cnt, pkb, msg)

            def ch_body(c, _2):
                cb = pl.multiple_of(c * CHUNK, CHUNK)
                load_unpack(j, tp, cb, pkb, idxb)
                pltpu.async_copy(table_h.at[idxb], msg, sem).wait()
                process(cnt - cb, pkb, msg)
                return 0

            lax.fori_loop(1, (cnt + CHUNK - 1) // CHUNK, ch_body, 0)

    def k_body(k, _):
        j = t + NTILES * k

        def init_body(i, _2):
            for f in range(D // 16):
                acc[i, pl.ds(f * 16, 16)] = neg
            return 0

        lax.fori_loop(0, MA, init_body, 0)
        pltpu.sync_copy(hdr_h.at[j], hdrv)
        fire(j, 0, pkbA, idxA, msgA, semA)

        def pair_body(i, _2):
            tp0 = i * 2
            fire(j, tp0 + 1, pkbB, idxB, msgB, semB)
            waitprocess(j, tp0, pkbA, idxA, msgA, semA)

            @pl.when(tp0 + 2 < NTILES)
            def _():
                fire(j, tp0 + 2, pkbA, idxA, msgA, semA)

            waitprocess(j, tp0 + 1, pkbB, idxB, msgB, semB)
            return 0

        lax.fori_loop(0, NTILES // 2, pair_body, 0)
        pltpu.sync_copy(acc.at[pl.ds(0, M)], hall_h.at[j])
        return 0

    lax.fori_loop(0, RPT, k_body, 0)


def _tc1_body(x_ref, hall_ref, root_ref, bias_ref, w_ref, out_ref):
    acc = jnp.dot(x_ref[...], root_ref[...],
                  preferred_element_type=jnp.float32) + bias_ref[...]
    hb = hall_ref[0]
    for r in range(R):
        h = hb[r * NR:(r + 1) * NR, :]
        h = jnp.where(h == NEG, 0.0, h)
        acc = acc + jnp.dot(h, w_ref[r], preferred_element_type=jnp.float32)
    out_ref[...] = jnp.maximum(acc, 0.0)


def _tc_layer1(xp, hall, root1, bias1, w1bd):
    return pl.pallas_call(
        _tc1_body,
        grid=(NRANGES,),
        in_specs=[
            pl.BlockSpec((NR, D), lambda j: (j, 0)),
            pl.BlockSpec((1, M, D), lambda j: (j, 0, 0)),
            pl.BlockSpec((D, D), lambda j: (0, 0)),
            pl.BlockSpec((1, D), lambda j: (0, 0)),
            pl.BlockSpec((R, D, D), lambda j: (0, 0, 0)),
        ],
        out_specs=pl.BlockSpec((NR, D), lambda j: (j, 0)),
        out_shape=jax.ShapeDtypeStruct((NPAD, D), jnp.float32),
    )(xp, hall, root1, bias1, w1bd)


def _tc2_body(h1_ref, hall_ref, root_ref, bias_ref, comp_ref, basis_ref,
              out_ref):
    acc = jnp.dot(h1_ref[...], root_ref[...],
                  preferred_element_type=jnp.float32) + bias_ref[...]
    hb = hall_ref[0]
    hfix = [None] * R
    for r in range(R):
        h = hb[r * NR:(r + 1) * NR, :]
        hfix[r] = jnp.where(h == NEG, 0.0, h)
    for b in range(4):
        g = hfix[0] * comp_ref[0, b]
        for r in range(1, R):
            g = g + hfix[r] * comp_ref[r, b]
        acc = acc + jnp.dot(g, basis_ref[b],
                            preferred_element_type=jnp.float32)
    out_ref[...] = acc


def _tc_layer2(h1, hall, root2, bias2, comp2, basis2):
    return pl.pallas_call(
        _tc2_body,
        grid=(NRANGES,),
        in_specs=[
            pl.BlockSpec((NR, D), lambda j: (j, 0)),
            pl.BlockSpec((1, M, D), lambda j: (j, 0, 0)),
            pl.BlockSpec((D, 2), lambda j: (0, 0)),
            pl.BlockSpec((1, 2), lambda j: (0, 0)),
            pl.BlockSpec((R, 4), lambda j: (0, 0)),
            pl.BlockSpec((4, D, 2), lambda j: (0, 0, 0)),
        ],
        out_specs=pl.BlockSpec((NR, 2), lambda j: (j, 0)),
        out_shape=jax.ShapeDtypeStruct((NPAD, 2), jnp.float32),
    )(h1, hall, root2, bias2, comp2, basis2)


@functools.cache
def _sc_kernels():
    bin_edges = _bin_kernel_def()(_bin_edges_body)
    segmax = _segmax_kernel_def()(_segmax_body)
    return bin_edges, segmax


def kernel(x, edge_index, edge_type, weight1, root1, bias1, comp2, basis2,
           root2, bias2):
    _bin_edges, _segmax = _sc_kernels()
    src = edge_index[0]
    dst = edge_index[1]
    et = edge_type.astype(jnp.int32)
    xp = jnp.pad(x, ((0, NPAD - N), (0, 0)))
    w1bd = jax.vmap(
        lambda w: jsl.block_diag(w[0], w[1], w[2], w[3]))(weight1)

    lists, hdr = _bin_edges(src, dst, et)
    hall1 = _segmax(xp, lists, hdr)
    h1 = _tc_layer1(xp, hall1, root1, bias1.reshape(1, D), w1bd)
    hall2 = _segmax(h1, lists, hdr)
    out = _tc_layer2(h1, hall2, root2, bias2.reshape(1, 2), comp2, basis2)
    return out[:N]


# speedup vs baseline: 8.4053x; 4.0311x over previous
import functools

import jax
import jax.numpy as jnp
import jax.scipy.linalg as jsl
from jax import lax
from jax.experimental import pallas as pl
from jax.experimental.pallas import tpu as pltpu
from jax.experimental.pallas import tpu_sc as plsc

N = 10000
E = 320000
D = 128
R = 8
NR = 64
NRANGES = 160
NPAD = NRANGES * NR
NTILES = 32
RPT = NRANGES // NTILES
EPT = E // NTILES
CAP = 10240
CE = 2000
CHUNK = 64
M = R * NR
DUMMY = M
MA = M + 16
NRP = 176
STG = EPT + NRANGES * (CHUNK - 1) + 32
NEG = float("-inf")


def _wid():
    return lax.axis_index("s") * 2 + lax.axis_index("c")


def _iota16():
    return lax.iota(jnp.int32, 16)


def _sstore(ref, i, val):
    b = pl.multiple_of(jnp.left_shift(jnp.right_shift(i, 3), 3), 8)
    w = ref[pl.ds(b, 16)]
    ref[pl.ds(b, 16)] = jnp.where(_iota16() == (i - b), val, w)


def _bin_kernel_def():
  return functools.partial(
    pl.kernel,
    mesh=plsc.VectorSubcoreMesh(core_axis_name="c", subcore_axis_name="s"),
    out_type=[
        jax.ShapeDtypeStruct((NRANGES, NTILES, CAP), jnp.int32),
        jax.ShapeDtypeStruct((NRANGES, NTILES, 16), jnp.int32),
    ],
    scratch_types=[
        pltpu.VMEM((CE,), jnp.int32),
        pltpu.VMEM((CE,), jnp.int32),
        pltpu.VMEM((CE,), jnp.int32),
        pltpu.VMEM((EPT,), jnp.int32),
        pltpu.VMEM((EPT,), jnp.int32),
        pltpu.SMEM((NRANGES,), jnp.int32),
        pltpu.SMEM((NRANGES,), jnp.int32),
        pltpu.SMEM((NRANGES,), jnp.int32),
        pltpu.VMEM((16,), jnp.int32),
        pltpu.VMEM((STG,), jnp.int32),
    ],
  )


def _bin_edges_body(src_h, dst_h, et_h, lists_h, hdr_h,
                    sbuf, dbuf, tbuf, ridb, pkb, cntv, offv, curv, hb, stg):
    t = _wid()
    base_e = pl.multiple_of(t * EPT, 8)

    def z_body(r, _):
        cntv[r] = 0
        return 0

    lax.fori_loop(0, NRANGES, z_body, 0)

    dum = jnp.full((16,), DUMMY, jnp.int32)

    def stg_body(v, _):
        stg[pl.ds(pl.multiple_of(v * 16, 16), 16)] = dum
        return 0

    lax.fori_loop(0, STG // 16, stg_body, 0)

    for c in range(EPT // CE):
        pltpu.sync_copy(src_h.at[pl.ds(base_e + c * CE, CE)], sbuf)
        pltpu.sync_copy(dst_h.at[pl.ds(base_e + c * CE, CE)], dbuf)
        pltpu.sync_copy(et_h.at[pl.ds(base_e + c * CE, CE)], tbuf)

        def v_body(v, _):
            vb = pl.multiple_of(v * 16, 16)
            d = dbuf[pl.ds(vb, 16)]
            s = sbuf[pl.ds(vb, 16)]
            ty = tbuf[pl.ds(vb, 16)]
            rid = jnp.right_shift(d, 6)
            pk = jnp.left_shift(s, 10) | jnp.left_shift(ty, 6) | (d & 63)
            g = pl.multiple_of(c * CE + vb, 16)
            ridb[pl.ds(g, 16)] = rid
            pkb[pl.ds(g, 16)] = pk
            return 0

        lax.fori_loop(0, CE // 16, v_body, 0)

    def cnt_body(v, _):
        vb = pl.multiple_of(v * 16, 16)
        rid = ridb[pl.ds(vb, 16)]
        for l in range(16):
            r = rid[l]
            cntv[r] = cntv[r] + 1
        return 0

    lax.fori_loop(0, EPT // 16, cnt_body, 0)

    def pfx_body(r, cum):
        offv[r] = cum
        curv[r] = cum
        return cum + ((cntv[r] + CHUNK - 1) & ~(CHUNK - 1))

    lax.fori_loop(0, NRANGES, pfx_body, jnp.int32(0))

    def app_body(v, _):
        vb = pl.multiple_of(v * 16, 16)
        rid = ridb[pl.ds(vb, 16)]
        pk = pkb[pl.ds(vb, 16)]
        for l in range(16):
            r = rid[l]
            o = curv[r]
            _sstore(stg, o, pk[l])
            curv[r] = o + 1
        return 0

    lax.fori_loop(0, EPT // 16, app_body, 0)

    def wr_body(r, _):
        cnt = cntv[r]
        o = offv[r]
        nch = (cnt + CHUNK - 1) // CHUNK

        def ch_body(c, _2):
            so = pl.multiple_of(o + c * CHUNK, CHUNK)
            do = pl.multiple_of(c * CHUNK, CHUNK)
            pltpu.sync_copy(stg.at[pl.ds(so, CHUNK)],
                            lists_h.at[r, t, pl.ds(do, CHUNK)])
            return 0

        lax.fori_loop(0, nch, ch_body, 0)
        hb[pl.ds(0, 16)] = jnp.where(_iota16() == 0, cnt, 0)
        pltpu.sync_copy(hb, hdr_h.at[r, t])
        return 0

    lax.fori_loop(0, NRANGES, wr_body, 0)


PR = 256
NSP = NPAD // PR
CAPT = 321536
RB = 512


def _segmax_kernel_def():
  return functools.partial(
    pl.kernel,
    mesh=plsc.VectorSubcoreMesh(core_axis_name="c", subcore_axis_name="s"),
    out_type=[
        jax.ShapeDtypeStruct((NRANGES, M, D), jnp.float32),
        jax.ShapeDtypeStruct((NTILES * CAPT,), jnp.int32),
    ],
    scratch_types=[
        pltpu.VMEM((MA, D), jnp.float32),
        pltpu.VMEM((PR, D), jnp.float32),
        pltpu.VMEM((NTILES, 16), jnp.int32),
        pltpu.VMEM((RB,), jnp.int32),
        pltpu.VMEM((NSP * 16,), jnp.int32),
        pltpu.SMEM((NSP,), jnp.int32),
        pltpu.SMEM((NSP + 8,), jnp.int32),
        pltpu.SMEM((NSP,), jnp.int32),
    ],
  )


def _segmax_body(table_h, lists_h, hdr_h, hall_h, lists2_h,
                 acc, piece, hdrv, rbuf, pbufs, cntp, offp, curp):
    t = _wid()
    neg = jnp.full((16,), NEG, jnp.float32)

    def getcnt(tp):
        return hdrv[tp, pl.ds(0, 16)][0]

    tb = pl.multiple_of(t * CAPT, 512)

    def k_body(k, _):
        j = t + NTILES * k

        def init_body(i, _2):
            for f in range(D // 16):
                acc[i, pl.ds(f * 16, 16)] = neg
            return 0

        lax.fori_loop(0, MA, init_body, 0)
        pltpu.sync_copy(hdr_h.at[j], hdrv)

        def zp_body(p, _2):
            cntp[p] = 0
            return 0

        lax.fori_loop(0, NSP, zp_body, 0)

        def h_tp_body(tp, _2):
            cnt = getcnt(tp)

            def h_ch_body(c, _3):
                cb = pl.multiple_of(c * RB, RB)
                pltpu.sync_copy(lists_h.at[j, tp, pl.ds(cb, RB)], rbuf)
                rem = jnp.minimum(cnt - cb, RB)

                def h_v_body(v, _4):
                    vb = pl.multiple_of(v * 16, 16)
                    pv = jnp.right_shift(rbuf[pl.ds(vb, 16)], 18)
                    for l in range(16):
                        p = pv[l]
                        cntp[p] = cntp[p] + 1
                    return 0

                lax.fori_loop(0, jnp.right_shift(rem + 15, 4), h_v_body, 0)
                return 0

            lax.fori_loop(0, (cnt + RB - 1) // RB, h_ch_body, 0)
            return 0

        lax.fori_loop(0, NTILES, h_tp_body, 0)

        def pfx_body(p, cum):
            offp[p] = cum
            curp[p] = cum
            return cum + ((cntp[p] + 15) & ~15)

        lax.fori_loop(0, NSP, pfx_body, jnp.int32(0))

        def s_tp_body(tp, _2):
            cnt = getcnt(tp)

            def s_ch_body(c, _3):
                cb = pl.multiple_of(c * RB, RB)
                pltpu.sync_copy(lists_h.at[j, tp, pl.ds(cb, RB)], rbuf)
                rem = jnp.minimum(cnt - cb, RB)

                def s_v_body(v, _4):
                    vb = pl.multiple_of(v * 16, 16)
                    pk = rbuf[pl.ds(vb, 16)]
                    pv = jnp.right_shift(pk, 18)
                    pk2 = (jnp.left_shift(jnp.right_shift(pk, 10) & 255, 10)
                           | (pk & 1023))
                    for l in range(16):
                        p = pv[l]
                        o = curp[p]
                        pb = pl.multiple_of(p * 16, 16)
                        w = pbufs[pl.ds(pb, 16)]
                        pbufs[pl.ds(pb, 16)] = jnp.where(
                            _iota16() == (o & 15), pk2[l], w)

                        @pl.when((o & 15) == 15)
                        def _():
                            fo = pl.multiple_of(o - 15, 16)
                            pltpu.sync_copy(
                                pbufs.at[pl.ds(pb, 16)],
                                lists2_h.at[pl.ds(tb + fo, 16)])

                        curp[p] = o + 1
                    return 0

                lax.fori_loop(0, jnp.right_shift(rem + 15, 4), s_v_body, 0)
                return 0

            lax.fori_loop(0, (cnt + RB - 1) // RB, s_ch_body, 0)
            return 0

        lax.fori_loop(0, NTILES, s_tp_body, 0)

        def pad_body(p, _2):
            o = curp[p]
            r16 = o & 15

            @pl.when(r16 > 0)
            def _():
                pb = pl.multiple_of(p * 16, 16)
                w = pbufs[pl.ds(pb, 16)]
                pbufs[pl.ds(pb, 16)] = jnp.where(
                    _iota16() < r16, w, DUMMY)
                fo = pl.multiple_of(o - r16, 16)
                pltpu.sync_copy(pbufs.at[pl.ds(pb, 16)],
                                lists2_h.at[pl.ds(tb + fo, 16)])
                curp[p] = o + 16 - r16

            return 0

        lax.fori_loop(0, NSP, pad_body, 0)

        def p_body(p, _2):
            lenp = curp[p] - offp[p]

            @pl.when(lenp > 0)
            def _():
                pltpu.sync_copy(table_h.at[p], piece)

                def c_body(c, _3):
                    cb = pl.multiple_of(tb + offp[p] + c * RB, 16)
                    pltpu.sync_copy(lists2_h.at[pl.ds(cb, RB)], rbuf)
                    rem = jnp.minimum(lenp - c * RB, RB)

                    def v_body(v, _4):
                        vb = pl.multiple_of(v * 16, 16)
                        pkv = rbuf[pl.ds(vb, 16)]
                        for l in range(16):
                            sl = jnp.right_shift(pkv[l], 10)
                            m = pkv[l] & 1023
                            for f in range(D // 16):
                                s2 = pl.ds(f * 16, 16)
                                acc[m, s2] = jnp.maximum(acc[m, s2],
                                                         piece[sl, s2])
                        return 0

                    lax.fori_loop(0, jnp.right_shift(rem, 4), v_body, 0)
                    return 0

                lax.fori_loop(0, (lenp + RB - 1) // RB, c_body, 0)

            return 0

        lax.fori_loop(0, NSP, p_body, 0)
        pltpu.sync_copy(acc.at[pl.ds(0, M)], hall_h.at[j])
        return 0

    lax.fori_loop(0, RPT, k_body, 0)


def _tc1_body(x_ref, hall_ref, root_ref, bias_ref, w_ref, out_ref):
    acc = jnp.dot(x_ref[...], root_ref[...],
                  preferred_element_type=jnp.float32) + bias_ref[...]
    hb = hall_ref[0]
    for r in range(R):
        h = hb[r * NR:(r + 1) * NR, :]
        h = jnp.where(h == NEG, 0.0, h)
        acc = acc + jnp.dot(h, w_ref[r], preferred_element_type=jnp.float32)
    out_ref[...] = jnp.maximum(acc, 0.0)


def _tc_layer1(xp, hall, root1, bias1, w1bd):
    return pl.pallas_call(
        _tc1_body,
        grid=(NRANGES,),
        in_specs=[
            pl.BlockSpec((NR, D), lambda j: (j, 0)),
            pl.BlockSpec((1, M, D), lambda j: (j, 0, 0)),
            pl.BlockSpec((D, D), lambda j: (0, 0)),
            pl.BlockSpec((1, D), lambda j: (0, 0)),
            pl.BlockSpec((R, D, D), lambda j: (0, 0, 0)),
        ],
        out_specs=pl.BlockSpec((NR, D), lambda j: (j, 0)),
        out_shape=jax.ShapeDtypeStruct((NPAD, D), jnp.float32),
    )(xp, hall, root1, bias1, w1bd)


def _tc2_body(h1_ref, hall_ref, root_ref, bias_ref, comp_ref, basis_ref,
              out_ref):
    acc = jnp.dot(h1_ref[...], root_ref[...],
                  preferred_element_type=jnp.float32) + bias_ref[...]
    hb = hall_ref[0]
    hfix = [None] * R
    for r in range(R):
        h = hb[r * NR:(r + 1) * NR, :]
        hfix[r] = jnp.where(h == NEG, 0.0, h)
    for b in range(4):
        g = hfix[0] * comp_ref[0, b]
        for r in range(1, R):
            g = g + hfix[r] * comp_ref[r, b]
        acc = acc + jnp.dot(g, basis_ref[b],
                            preferred_element_type=jnp.float32)
    out_ref[...] = acc


def _tc_layer2(h1, hall, root2, bias2, comp2, basis2):
    return pl.pallas_call(
        _tc2_body,
        grid=(NRANGES,),
        in_specs=[
            pl.BlockSpec((NR, D), lambda j: (j, 0)),
            pl.BlockSpec((1, M, D), lambda j: (j, 0, 0)),
            pl.BlockSpec((D, 2), lambda j: (0, 0)),
            pl.BlockSpec((1, 2), lambda j: (0, 0)),
            pl.BlockSpec((R, 4), lambda j: (0, 0)),
            pl.BlockSpec((4, D, 2), lambda j: (0, 0, 0)),
        ],
        out_specs=pl.BlockSpec((NR, 2), lambda j: (j, 0)),
        out_shape=jax.ShapeDtypeStruct((NPAD, 2), jnp.float32),
    )(h1, hall, root2, bias2, comp2, basis2)


@functools.cache
def _sc_kernels():
    bin_edges = _bin_kernel_def()(_bin_edges_body)
    segmax = _segmax_kernel_def()(_segmax_body)
    return bin_edges, segmax


def kernel(x, edge_index, edge_type, weight1, root1, bias1, comp2, basis2,
           root2, bias2):
    _bin_edges, _segmax = _sc_kernels()
    src = edge_index[0]
    dst = edge_index[1]
    et = edge_type.astype(jnp.int32)
    xp = jnp.pad(x, ((0, NPAD - N), (0, 0)))
    w1bd = jax.vmap(
        lambda w: jsl.block_diag(w[0], w[1], w[2], w[3]))(weight1)

    lists, hdr = _bin_edges(src, dst, et)
    hall1, _scr1 = _segmax(xp.reshape(NSP, PR, D), lists, hdr)
    h1 = _tc_layer1(xp, hall1, root1, bias1.reshape(1, D), w1bd)
    hall2, _scr2 = _segmax(h1.reshape(NSP, PR, D), lists, hdr)
    out = _tc_layer2(h1, hall2, root2, bias2.reshape(1, 2), comp2, basis2)
    return out[:N]
